# baseline (speedup 1.0000x reference)
import functools

import jax
import jax.numpy as jnp
from jax import lax
from jax.experimental import pallas as pl
from jax.experimental.pallas import tpu as pltpu
from jax.experimental.pallas import tpu_sc as plsc

N_FAMILY = 100000
DIM_FAMILY = 128
N_CHAR = 1000
DIM_CHAR = 64
BATCH = 4096
DIM_OUT = DIM_FAMILY + DIM_CHAR

NUM_CORES = 2
NUM_SUBCORES = 16
NUM_WORKERS = NUM_CORES * NUM_SUBCORES
BW = BATCH // NUM_WORKERS
DT = DIM_OUT // 8

_mesh = plsc.VectorSubcoreMesh(core_axis_name="c", subcore_axis_name="s")


@functools.partial(
    pl.kernel,
    mesh=_mesh,
    compiler_params=pltpu.CompilerParams(use_tc_tiling_on_sc=False,
                                         needs_layout_passes=False),
    out_type=jax.ShapeDtypeStruct((DT, NUM_WORKERS, 8, BW), jnp.float32),
    scratch_types=[
        pltpu.VMEM((BW,), jnp.int32),
        pltpu.VMEM((BW,), jnp.int32),
        pltpu.VMEM((BW, DIM_FAMILY), jnp.float32),
        pltpu.VMEM((BW, DIM_CHAR), jnp.float32),
        pltpu.VMEM((DT, 1, 8, BW + 1), jnp.float32),
        pltpu.SemaphoreType.DMA,
        pltpu.SemaphoreType.DMA,
        pltpu.SemaphoreType.DMA,
    ],
)
def _embed(font_hbm, char_hbm, fam_tab, chr_tab, out_hbm,
           fidx_v, cidx_v, fam_v, chr_v, tr_v, sem_i, sem_f, sem_c):
    wid = lax.axis_index("s") * NUM_CORES + lax.axis_index("c")
    base = wid * BW
    cp_fi = pltpu.async_copy(font_hbm.at[pl.ds(base, BW)], fidx_v, sem_i)
    cp_ci = pltpu.async_copy(char_hbm.at[pl.ds(base, BW)], cidx_v, sem_i)
    cp_fi.wait()
    cp_f = pltpu.async_copy(fam_tab.at[fidx_v], fam_v, sem_f)
    cp_ci.wait()
    cp_c = pltpu.async_copy(chr_tab.at[cidx_v], chr_v, sem_c)

    lane = lax.iota(jnp.int32, 16)
    zero16 = jnp.zeros((16,), jnp.int32)
    fam_dt = [(lane + (j * 16)) // 8 for j in range(DIM_FAMILY // 16)]
    fam_dr = [(lane + (j * 16)) % 8 for j in range(DIM_FAMILY // 16)]
    chr_dt = [(lane + (DIM_FAMILY + j * 16)) // 8 for j in range(DIM_CHAR // 16)]
    chr_dr = [(lane + (j * 16)) % 8 for j in range(DIM_CHAR // 16)]

    def tr_row(b, _):
        bvec = jnp.full((16,), b, jnp.int32)
        for j in range(DIM_FAMILY // 16):
            v = fam_v[b, pl.ds(j * 16, 16)]
            plsc.store_scatter(tr_v, [fam_dt[j], zero16, fam_dr[j], bvec], v)
        for j in range(DIM_CHAR // 16):
            v = chr_v[b, pl.ds(j * 16, 16)]
            plsc.store_scatter(tr_v, [chr_dt[j], zero16, chr_dr[j], bvec], v)
        return 0

    cp_f.wait()
    cp_c.wait()
    lax.fori_loop(0, BW, tr_row, 0)
    pltpu.sync_copy(tr_v.at[:, :, :, pl.ds(0, BW)],
                    out_hbm.at[:, pl.ds(wid, 1), :, :])


def kernel(font_idx, char_idx, family_table, char_table):
    out4 = _embed(font_idx.astype(jnp.int32), char_idx.astype(jnp.int32),
                  family_table, char_table)
    return jnp.transpose(out4, (1, 3, 0, 2)).reshape(BATCH, DIM_OUT)


# speedup vs baseline: 1.1882x; 1.1882x over previous
import functools

import jax
import jax.numpy as jnp
from jax import lax
from jax.experimental import pallas as pl
from jax.experimental.pallas import tpu as pltpu
from jax.experimental.pallas import tpu_sc as plsc

N_FAMILY = 100000
DIM_FAMILY = 128
N_CHAR = 1000
DIM_CHAR = 64
BATCH = 4096
DIM_OUT = DIM_FAMILY + DIM_CHAR

NUM_CORES = 2
NUM_SUBCORES = 16
NUM_WORKERS = NUM_CORES * NUM_SUBCORES
BW = BATCH // NUM_WORKERS
DT = DIM_OUT // 8

_mesh = plsc.VectorSubcoreMesh(core_axis_name="c", subcore_axis_name="s")


@functools.partial(
    pl.kernel,
    mesh=_mesh,
    compiler_params=pltpu.CompilerParams(use_tc_tiling_on_sc=False,
                                         needs_layout_passes=False),
    out_type=jax.ShapeDtypeStruct((DT, NUM_WORKERS, 8, BW), jnp.float32),
    scratch_types=[
        pltpu.VMEM((BW,), jnp.int32),
        pltpu.VMEM((BW,), jnp.int32),
        pltpu.VMEM((BW, DIM_FAMILY), jnp.float32),
        pltpu.VMEM((BW, DIM_CHAR), jnp.float32),
        pltpu.VMEM((DT, 1, 8, BW + 1), jnp.float32),
        pltpu.SemaphoreType.DMA,
        pltpu.SemaphoreType.DMA,
        pltpu.SemaphoreType.DMA,
    ],
)
def _embed(font_hbm, char_hbm, fam_tab, chr_tab, out_hbm,
           fidx_v, cidx_v, fam_v, chr_v, tr_v, sem_i, sem_f, sem_c):
    wid = lax.axis_index("s") * NUM_CORES + lax.axis_index("c")
    base = wid * BW
    cp_fi = pltpu.async_copy(font_hbm.at[pl.ds(base, BW)], fidx_v, sem_i)
    cp_ci = pltpu.async_copy(char_hbm.at[pl.ds(base, BW)], cidx_v, sem_i)
    cp_fi.wait()
    cp_f = pltpu.async_copy(fam_tab.at[fidx_v], fam_v, sem_f)
    cp_ci.wait()
    cp_c = pltpu.async_copy(chr_tab.at[cidx_v], chr_v, sem_c)

    lane = lax.iota(jnp.int32, 16)
    zero16 = jnp.zeros((16,), jnp.int32)
    fam_dt = [(lane + (j * 16)) // 8 for j in range(DIM_FAMILY // 16)]
    fam_dr = [(lane + (j * 16)) % 8 for j in range(DIM_FAMILY // 16)]
    chr_dt = [(lane + (DIM_FAMILY + j * 16)) // 8 for j in range(DIM_CHAR // 16)]
    chr_dr = [(lane + (j * 16)) % 8 for j in range(DIM_CHAR // 16)]

    cp_f.wait()
    cp_c.wait()

    @plsc.parallel_loop(0, BW)
    def tr_row(b):
        bvec = jnp.full((16,), b, jnp.int32)
        for j in range(DIM_FAMILY // 16):
            v = fam_v[b, pl.ds(j * 16, 16)]
            plsc.store_scatter(tr_v, [fam_dt[j], zero16, fam_dr[j], bvec], v)
        for j in range(DIM_CHAR // 16):
            v = chr_v[b, pl.ds(j * 16, 16)]
            plsc.store_scatter(tr_v, [chr_dt[j], zero16, chr_dr[j], bvec], v)
    pltpu.sync_copy(tr_v.at[:, :, :, pl.ds(0, BW)],
                    out_hbm.at[:, pl.ds(wid, 1), :, :])


def kernel(font_idx, char_idx, family_table, char_table):
    out4 = _embed(font_idx.astype(jnp.int32), char_idx.astype(jnp.int32),
                  family_table, char_table)
    return jnp.transpose(out4, (1, 3, 0, 2)).reshape(BATCH, DIM_OUT)
